# baseline (speedup 1.0000x reference)
import functools

import jax
import jax.numpy as jnp
from jax import lax
from jax.experimental import pallas as pl
from jax.experimental.pallas import tpu as pltpu
from jax.experimental.pallas import tpu_sc as plsc

NN_SIZE = 16
RADIUS2 = 0.05 * 0.05

B, N, C = 8, 16384, 3

NC = 2
NS = 16
L = 16
NW = NC * NS
WPB = NW // B
R = N // WPB
GROUPS = R // L
TILE = 8 * 128
KT = NN_SIZE // 8
NTW = R // 128


def _rsqrt(x):
    i = plsc.bitcast(x, jnp.int32)
    i = jnp.int32(0x5F3759DF) - lax.shift_right_logical(i, 1)
    y = plsc.bitcast(i, jnp.float32)
    return y * (1.5 - 0.5 * x * y * y)


@functools.partial(
    pl.kernel,
    mesh=plsc.VectorSubcoreMesh(core_axis_name="c", subcore_axis_name="s"),
    compiler_params=pltpu.CompilerParams(needs_layout_passes=False),
    out_type=jax.ShapeDtypeStruct((NW * L,), jnp.float32),
    scratch_types=[
        pltpu.VMEM((N,), jnp.float32),
        pltpu.VMEM((N,), jnp.float32),
        pltpu.VMEM((N,), jnp.float32),
        pltpu.VMEM((NN_SIZE * R,), jnp.int32),
        pltpu.VMEM((L,), jnp.float32),
        pltpu.SemaphoreType.DMA,
        pltpu.SemaphoreType.DMA,
        pltpu.SemaphoreType.DMA,
    ],
)
def _repulsion_sc(pts_hbm, idx_hbm, out_hbm, x_v, y_v, z_v, idx_v, acc_v, sem,
                  sem_i0, sem_i1):
    wid = lax.axis_index("s") * NC + lax.axis_index("c")
    b = wid // WPB
    q = wid % WPB
    base = q * R

    copies = [
        pltpu.async_copy(
            pts_hbm.at[pl.ds((c * (N // 128) + nt) * (B * 128) + b * 128, 128)],
            dst.at[pl.ds(nt * 128, 128)],
            sem,
        )
        for c, dst in zip(range(C), (x_v, y_v, z_v))
        for nt in range(N // 128)
    ]
    idx_copies = [
        pltpu.async_copy(
            idx_hbm.at[pl.ds(((b * KT + kt) * (N // 128) + q * NTW) * TILE,
                             NTW * TILE)],
            idx_v.at[pl.ds(kt * NTW * TILE, NTW * TILE)],
            isem,
        )
        for kt, isem in zip(range(KT), (sem_i0, sem_i1))
    ]

    def make_body(ks):
        def body(g, acc):
            g16 = g * L
            grp = lax.shift_left((g16 >> 7), 10) + (g16 & 127)
            cx = x_v[pl.ds(base + g16, L)]
            cy = y_v[pl.ds(base + g16, L)]
            cz = z_v[pl.ds(base + g16, L)]
            for k in ks:
                koff = (k // 8) * (NTW * TILE) + (k % 8) * 128
                nidx = idx_v[pl.ds(grp + koff, L)]
                dx = plsc.load_gather(x_v, [nidx]) - cx
                dy = plsc.load_gather(y_v, [nidx]) - cy
                dz = plsc.load_gather(z_v, [nidx]) - cz
                d2 = (dx * dx + dy * dy) + dz * dz
                acc = acc + jnp.where(d2 < RADIUS2, _rsqrt(d2 + 0.0001), 0.0)
            return acc

        return body

    for c in copies:
        c.wait()
    idx_copies[0].wait()
    acc = lax.fori_loop(0, GROUPS, make_body(range(8)),
                        jnp.zeros((L,), jnp.float32))
    idx_copies[1].wait()
    acc = lax.fori_loop(0, GROUPS, make_body(range(8, NN_SIZE)), acc)
    acc_v[...] = acc
    pltpu.sync_copy(acc_v, out_hbm.at[pl.ds(wid * L, L)])


def kernel(points, knn_idx):
    pts_t = (
        jnp.transpose(points, (2, 0, 1))
        .reshape(C, B, N // 128, 128)
        .transpose(0, 2, 1, 3)
        .reshape(C * B * N)
    )
    idx_t = (
        jnp.swapaxes(knn_idx, 1, 2)
        .reshape(B, KT, 8, N // 128, 128)
        .transpose(0, 1, 3, 2, 4)
        .reshape(B * NN_SIZE * N)
    )
    partials = _repulsion_sc(pts_t, idx_t)
    return jnp.sum(partials) / (B * N * NN_SIZE)


# speedup vs baseline: 1.1134x; 1.1134x over previous
import functools

import jax
import jax.numpy as jnp
from jax import lax
from jax.experimental import pallas as pl
from jax.experimental.pallas import tpu as pltpu
from jax.experimental.pallas import tpu_sc as plsc

NN_SIZE = 16
RADIUS2 = 0.05 * 0.05

B, N, C = 8, 16384, 3

NC = 2
NS = 16
L = 16
NW = NC * NS
WPB = NW // B
R = N // WPB
GROUPS = R // L
TILE = 8 * 128
KT = NN_SIZE // 8
NTW = R // 128


def _rsqrt(x):
    i = plsc.bitcast(x, jnp.int32)
    i = jnp.int32(0x5F3759DF) - lax.shift_right_logical(i, 1)
    y = plsc.bitcast(i, jnp.float32)
    return y * (1.5 - 0.5 * x * y * y)


@functools.partial(
    pl.kernel,
    mesh=plsc.VectorSubcoreMesh(core_axis_name="c", subcore_axis_name="s"),
    compiler_params=pltpu.CompilerParams(needs_layout_passes=False),
    out_type=jax.ShapeDtypeStruct((NW * L,), jnp.float32),
    scratch_types=[
        pltpu.VMEM((N,), jnp.float32),
        pltpu.VMEM((N,), jnp.float32),
        pltpu.VMEM((N,), jnp.float32),
        pltpu.VMEM((NN_SIZE * R,), jnp.int32),
        pltpu.VMEM((L,), jnp.float32),
        pltpu.SemaphoreType.DMA,
        pltpu.SemaphoreType.DMA,
        pltpu.SemaphoreType.DMA,
    ],
)
def _repulsion_sc(pts_hbm, idx_hbm, out_hbm, x_v, y_v, z_v, idx_v, acc_v, sem,
                  sem_i0, sem_i1):
    wid = lax.axis_index("s") * NC + lax.axis_index("c")
    b = wid // WPB
    q = wid % WPB
    base = q * R

    copies = [
        pltpu.async_copy(pts_hbm.at[pl.ds(c * B * N + b * N, N)], dst, sem)
        for c, dst in zip(range(C), (x_v, y_v, z_v))
    ]
    idx_copies = [
        pltpu.async_copy(
            idx_hbm.at[pl.ds(((b * KT + kt) * (N // 128) + q * NTW) * TILE,
                             NTW * TILE)],
            idx_v.at[pl.ds(kt * NTW * TILE, NTW * TILE)],
            isem,
        )
        for kt, isem in zip(range(KT), (sem_i0, sem_i1))
    ]

    def make_body(ks):
        def body(g, acc):
            g16 = g * L
            grp = lax.shift_left((g16 >> 7), 10) + (g16 & 127)
            cx = x_v[pl.ds(base + g16, L)]
            cy = y_v[pl.ds(base + g16, L)]
            cz = z_v[pl.ds(base + g16, L)]
            for k in ks:
                koff = (k // 8) * (NTW * TILE) + (k % 8) * 128
                nidx = idx_v[pl.ds(grp + koff, L)]
                dx = plsc.load_gather(x_v, [nidx]) - cx
                dy = plsc.load_gather(y_v, [nidx]) - cy
                dz = plsc.load_gather(z_v, [nidx]) - cz
                d2 = (dx * dx + dy * dy) + dz * dz
                acc = acc + jnp.where(d2 < RADIUS2, _rsqrt(d2 + 0.0001), 0.0)
            return acc

        return body

    for c in copies:
        c.wait()
    idx_copies[0].wait()
    acc = lax.fori_loop(0, GROUPS, make_body(range(8)),
                        jnp.zeros((L,), jnp.float32))
    idx_copies[1].wait()
    acc = lax.fori_loop(0, GROUPS, make_body(range(8, NN_SIZE)), acc)
    acc_v[...] = acc
    pltpu.sync_copy(acc_v, out_hbm.at[pl.ds(wid * L, L)])


def kernel(points, knn_idx):
    pts_t = jnp.transpose(points, (2, 0, 1)).reshape(C * B * N)
    idx_t = (
        jnp.swapaxes(knn_idx, 1, 2)
        .reshape(B, KT, 8, N // 128, 128)
        .transpose(0, 1, 3, 2, 4)
        .reshape(B * NN_SIZE * N)
    )
    partials = _repulsion_sc(pts_t, idx_t)
    return jnp.sum(partials) / (B * N * NN_SIZE)
